# baseline (speedup 1.0000x reference)
import functools

import jax
import jax.numpy as jnp
from jax import lax
from jax.experimental import pallas as pl
from jax.experimental.pallas import tpu as pltpu
from jax.experimental.pallas import tpu_sc as plsc

B = 2048
D = 768
SAE = 6144
K = 3
EMA_COEFF = 0.01

NC = 2
NS = 16
L = 16
NW = NC * NS
ROWS_PER_W = B // NW
RCH = 8
ECH = RCH * K
NCHUNK = ROWS_PER_W // RCH

BR = 256
BBIN = 128
NEG = -3.0e38


def _topk_body(x_ref, w_ref, b_ref, vals_ref, idx_ref):
    pre = lax.dot_general(
        x_ref[...], w_ref[...], (((1,), (1,)), ((), ())),
        preferred_element_type=jnp.float32,
    ) + b_ref[...]
    iota = lax.broadcasted_iota(jnp.int32, pre.shape, 1)
    v = pre
    for k in range(K):
        m = jnp.max(v, axis=1, keepdims=True)
        cand = jnp.where(v >= m, iota, SAE)
        am = jnp.min(cand, axis=1, keepdims=True)
        vals_ref[:, k:k + 1] = m
        idx_ref[:, k:k + 1] = am
        if k + 1 < K:
            v = jnp.where(iota == am, NEG, v)


def _run_topk(x, w_enc, b_enc):
    vals_p, idx_p = pl.pallas_call(
        _topk_body,
        grid=(B // BR,),
        in_specs=[
            pl.BlockSpec((BR, D), lambda i: (i, 0)),
            pl.BlockSpec((SAE, D), lambda i: (0, 0)),
            pl.BlockSpec((1, SAE), lambda i: (0, 0)),
        ],
        out_specs=[
            pl.BlockSpec((BR, 128), lambda i: (i, 0)),
            pl.BlockSpec((BR, 128), lambda i: (i, 0)),
        ],
        out_shape=[
            jax.ShapeDtypeStruct((B, 128), jnp.float32),
            jax.ShapeDtypeStruct((B, 128), jnp.int32),
        ],
    )(x, w_enc, b_enc.reshape(1, SAE))
    return vals_p[:, :K], idx_p[:, :K]


GPB = 8
AROWS = SAE // GPB
AL = GPB * L
SROWS = AROWS // NS


def _sc_body(x_hbm, idx_hbm, vals_hbm,
             we1_hbm, be1_hbm, we2_hbm, be2_hbm,
             wd0_hbm, wd1_hbm, wd2_hbm, bias_hbm,
             recon_hbm, acc_hbm,
             idx_v, idxp_v, idx8_v, vals_v, b1_v, b2_v,
             e1_v, e2_v, d0_v, d1_v, d2_v,
             x_v, out_v, bias_v,
             cr0_v, cr1_v, cr2_v,
             stat_v, zbuf_v, acc_sh,
             sem, semd, semb, semdb):
    cid = lax.axis_index("c")
    sid = lax.axis_index("s")
    wid = sid * NC + cid
    row0 = wid * ROWS_PER_W
    lane = lax.broadcasted_iota(jnp.int32, (L,), 0)
    zvec = jnp.zeros((L,), jnp.float32)

    def zstep(i, _):
        for gg in range(GPB):
            zbuf_v[i, pl.ds(gg * L, L)] = zvec
        return 0
    lax.fori_loop(0, SROWS, zstep, 0)
    pltpu.sync_copy(zbuf_v, acc_sh.at[pl.ds(sid * SROWS, SROWS)])
    plsc.subcore_barrier()

    pltpu.sync_copy(bias_hbm, bias_v)

    def _sget(ref, e):
        g = (e // L) * L
        return ref[pl.ds(g, L)][e - g]

    def _rbf(v):
        u = lax.bitcast_convert_type(v, jnp.int32)
        r = (u + 0x8000) & jnp.int32(-65536)
        return lax.bitcast_convert_type(r, jnp.float32)

    def chunk_body(c, carry):
        rbase = row0 + c * RCH
        ebase = rbase * K
        pltpu.sync_copy(idx_hbm.at[pl.ds(ebase, ECH)], idx_v)
        pltpu.sync_copy(idx_hbm.at[pl.ds(ebase, ECH)], idxp_v.at[pl.ds(0, ECH)])
        pltpu.sync_copy(vals_hbm.at[pl.ds(ebase, ECH)], vals_v.at[pl.ds(0, ECH)])
        idx8_v[pl.ds(0, L)] = lax.shift_right_logical(idxp_v[pl.ds(0, L)], 3)
        idx8_v[pl.ds(ECH - L, L)] = lax.shift_right_logical(
            idxp_v[pl.ds(ECH - L, L)], 3)
        EA, EB = 16, 8
        RA = 5
        idxA = idx_v.at[pl.ds(0, EA)]
        idxB = idx_v.at[pl.ds(EA, EB)]
        enc_a = [
            pltpu.async_copy(we1_hbm.at[idxA], e1_v.at[pl.ds(0, EA)], sem),
            pltpu.async_copy(we2_hbm.at[idxA], e2_v.at[pl.ds(0, EA)], sem),
            pltpu.async_copy(be1_hbm.at[idx_v], b1_v.at[pl.ds(0, ECH)], sem),
            pltpu.async_copy(be2_hbm.at[idx_v], b2_v.at[pl.ds(0, ECH)], sem),
            pltpu.async_copy(x_hbm.at[pl.ds(rbase, RCH)], x_v, sem),
        ]
        dec_a = [
            pltpu.async_copy(wd0_hbm.at[idxA], d0_v.at[pl.ds(0, EA)], semd),
            pltpu.async_copy(wd1_hbm.at[idxA], d1_v.at[pl.ds(0, EA)], semd),
            pltpu.async_copy(wd2_hbm.at[idxA], d2_v.at[pl.ds(0, EA)], semd),
        ]
        enc_b = [
            pltpu.async_copy(we1_hbm.at[idxB], e1_v.at[pl.ds(EA, EB)], semb),
            pltpu.async_copy(we2_hbm.at[idxB], e2_v.at[pl.ds(EA, EB)], semb),
        ]
        dec_b = [
            pltpu.async_copy(wd0_hbm.at[idxB], d0_v.at[pl.ds(EA, EB)], semdb),
            pltpu.async_copy(wd1_hbm.at[idxB], d1_v.at[pl.ds(EA, EB)], semdb),
            pltpu.async_copy(wd2_hbm.at[idxB], d2_v.at[pl.ds(EA, EB)], semdb),
        ]

        def entry_phase(e):
            r = e // K

            def dstep(i, accs):
                a1, a2 = accs
                xs = _rbf(x_v[r, pl.ds(i * L, L)])
                return (a1 + _rbf(e1_v[e, pl.ds(i * L, L)]) * xs,
                        a2 + _rbf(e2_v[e, pl.ds(i * L, L)]) * xs)

            z = jnp.zeros((L,), jnp.float32)
            a1, a2 = lax.fori_loop(0, D // L, dstep, (z, z))
            p1 = jnp.sum(a1) + _sget(b1_v, e)
            p2 = jnp.sum(a2) + _sget(b2_v, e)
            v = _sget(vals_v, e)
            nz = v != 0.0
            m1 = jnp.where(nz, p1, 0.0)
            m2 = jnp.where(nz, p2, 0.0)
            w = m1 > m2
            f1 = jnp.where(w, m1, 0.0)
            f2 = jnp.where(w, 0.0, m2)
            cr0_v[e, :] = _rbf(jnp.full((L,), v))
            cr1_v[e, :] = _rbf(jnp.full((L,), f1))
            cr2_v[e, :] = _rbf(jnp.full((L,), f2))
            sv = jnp.full((L,), jnp.where(nz, v, 1.0))
            f1v = jnp.full((L,), f1)
            f2v = jnp.full((L,), f2)
            c1v = jnp.where(f1v != 0.0, 1.0, 0.0)
            c2v = jnp.where(f2v != 0.0, 1.0, 0.0)
            num = jnp.where(lane == 2, f1v, jnp.where(lane == 4, f2v, 0.0))
            cnts = jnp.where(lane == 0, 1.0,
                   jnp.where(lane == 1, c1v,
                   jnp.where(lane == 3, c2v, 0.0)))
            stat = cnts + num / sv
            g = _sget(idxp_v, e) & (GPB - 1)
            for gg in range(GPB):
                stat_v[e, pl.ds(gg * L, L)] = jnp.where(g == gg, stat, zvec)

        def recon_row(r):
            coefs = []
            for k in range(K):
                e = r * K + k
                coefs.append((cr0_v[e, :], cr1_v[e, :], cr2_v[e, :]))

            def rstep(i, _):
                acc = bias_v[pl.ds(i * L, L)]
                for k in range(K):
                    e = r * K + k
                    cc0, cc1, cc2 = coefs[k]
                    acc = acc + cc0 * _rbf(d0_v[e, pl.ds(i * L, L)])
                    acc = acc + cc1 * _rbf(d1_v[e, pl.ds(i * L, L)])
                    acc = acc + cc2 * _rbf(d2_v[e, pl.ds(i * L, L)])
                out_v[r, pl.ds(i * L, L)] = acc
                return 0

            lax.fori_loop(0, D // L, rstep, 0)

        for cp in enc_a:
            cp.wait()
        for e in range(EA):
            entry_phase(e)
        for cp in dec_a:
            cp.wait()
        for r in range(RA):
            recon_row(r)
        for cp in enc_b:
            cp.wait()
        for e in range(EA, ECH):
            entry_phase(e)
        for cp in dec_b:
            cp.wait()
        for r in range(RA, RCH):
            recon_row(r)

        pltpu.sync_copy(out_v, recon_hbm.at[pl.ds(rbase, RCH)])
        pltpu.sync_copy(stat_v, acc_sh.at[idx8_v], add=True)
        return carry

    lax.fori_loop(0, NCHUNK, chunk_body, 0)

    plsc.subcore_barrier()
    pltpu.sync_copy(acc_sh.at[pl.ds(sid * SROWS, SROWS)],
                    acc_hbm.at[cid, pl.ds(sid * SROWS, SROWS)])


def _run_sc(x, idx_flat, vals_flat, w_enc1, b_enc1, w_enc2, b_enc2,
            wd0t, wd1t, wd2t, bias_sum):
    mesh = plsc.VectorSubcoreMesh(core_axis_name="c", subcore_axis_name="s",
                                  num_cores=NC, num_subcores=NS)
    f = pl.kernel(
        _sc_body,
        out_type=[
            jax.ShapeDtypeStruct((B, D), jnp.float32),
            jax.ShapeDtypeStruct((NC, AROWS, AL), jnp.float32),
        ],
        mesh=mesh,
        scratch_types=[
            pltpu.VMEM((ECH,), jnp.int32),
            pltpu.VMEM((ECH + L,), jnp.int32),
            pltpu.VMEM((ECH,), jnp.int32),
            pltpu.VMEM((ECH + L,), jnp.float32),
            pltpu.VMEM((ECH + L,), jnp.float32),
            pltpu.VMEM((ECH + L,), jnp.float32),
            pltpu.VMEM((ECH, D), jnp.float32),
            pltpu.VMEM((ECH, D), jnp.float32),
            pltpu.VMEM((ECH, D), jnp.float32),
            pltpu.VMEM((ECH, D), jnp.float32),
            pltpu.VMEM((ECH, D), jnp.float32),
            pltpu.VMEM((RCH, D), jnp.float32),
            pltpu.VMEM((RCH, D), jnp.float32),
            pltpu.VMEM((D,), jnp.float32),
            pltpu.VMEM((ECH, L), jnp.float32),
            pltpu.VMEM((ECH, L), jnp.float32),
            pltpu.VMEM((ECH, L), jnp.float32),
            pltpu.VMEM((ECH, AL), jnp.float32),
            pltpu.VMEM((SROWS, AL), jnp.float32),
            pltpu.VMEM_SHARED((AROWS, AL), jnp.float32),
            pltpu.SemaphoreType.DMA,
            pltpu.SemaphoreType.DMA,
            pltpu.SemaphoreType.DMA,
            pltpu.SemaphoreType.DMA,
        ],
        compiler_params=pltpu.CompilerParams(needs_layout_passes=False),
    )
    return f(x, idx_flat, vals_flat, w_enc1, b_enc1, w_enc2, b_enc2,
             wd0t, wd1t, wd2t, bias_sum)


def _finale_body(a0_ref, a1_ref, r1_ref, r2_ref,
                 r1o_ref, r2o_ref, live_ref):
    s = a0_ref[...] + a1_ref[...]
    cnt_p = s[:, 0:1]
    cnt1 = s[:, 1:2]
    sum1 = s[:, 2:3]
    cnt2 = s[:, 3:4]
    sum2 = s[:, 4:5]

    def ema(cnt, ssum, old):
        mean = ssum / jnp.maximum(cnt, 1.0)
        upd = (1.0 - EMA_COEFF * cnt) * old + EMA_COEFF * mean * cnt
        return jnp.where(cnt > 0.0, upd, old)

    r1o_ref[...] = ema(cnt1, sum1, r1_ref[...])
    r2o_ref[...] = ema(cnt2, sum2, r2_ref[...])

    lp = jnp.sum((cnt_p > 0.0).astype(jnp.float32))
    l1 = jnp.sum((cnt1 > 0.0).astype(jnp.float32))
    l2 = jnp.sum((cnt2 > 0.0).astype(jnp.float32))
    lanevec = lax.broadcasted_iota(jnp.int32, (1, 128), 1)
    live_ref[...] = jnp.where(lanevec == 0, lp,
                              jnp.where(lanevec == 1, l1,
                                        jnp.where(lanevec == 2, l2, 0.0)))


def _run_finale(acc, ratios1, ratios2):
    r1o, r2o, live = pl.pallas_call(
        _finale_body,
        out_shape=[
            jax.ShapeDtypeStruct((SAE, 1), jnp.float32),
            jax.ShapeDtypeStruct((SAE, 1), jnp.float32),
            jax.ShapeDtypeStruct((1, 128), jnp.float32),
        ],
    )(acc[0], acc[1], ratios1.reshape(SAE, 1), ratios2.reshape(SAE, 1))
    return r1o, r2o, live


def kernel(model_activations, W_enc, b_enc, W_dec, b_dec,
           W_enc1, b_enc1, W_dec1, b_dec1,
           W_enc2, b_enc2, W_dec2, b_dec2,
           child1_parent_ratios, child2_parent_ratios):
    x = model_activations
    topk_vals, topk_idx = _run_topk(x, W_enc, b_enc)
    idx_flat = topk_idx.reshape(-1)
    vals_flat = topk_vals.reshape(-1)

    wd0t = W_dec.T
    wd1t = W_dec1.T
    wd2t = W_dec2.T
    bias_sum = b_dec + b_dec1 + b_dec2

    recon, acc = _run_sc(
        x, idx_flat, vals_flat, W_enc1, b_enc1, W_enc2, b_enc2,
        wd0t, wd1t, wd2t, bias_sum)

    r1o, r2o, live = _run_finale(acc.reshape(NC, SAE, L),
                                 child1_parent_ratios,
                                 child2_parent_ratios)

    live_counts = live[0, :3].astype(jnp.int32)
    return (recon, live_counts,
            r1o.reshape(SAE), r2o.reshape(SAE))


# speedup vs baseline: 1.0058x; 1.0058x over previous
import functools

import jax
import jax.numpy as jnp
from jax import lax
from jax.experimental import pallas as pl
from jax.experimental.pallas import tpu as pltpu
from jax.experimental.pallas import tpu_sc as plsc

B = 2048
D = 768
SAE = 6144
K = 3
EMA_COEFF = 0.01

NC = 2
NS = 16
L = 16
NW = NC * NS
ROWS_PER_W = B // NW
RCH = 8
ECH = RCH * K
NCHUNK = ROWS_PER_W // RCH

BR = 256
BBIN = 128
NEG = -3.0e38


def _topk_body(x_ref, w_ref, b_ref, vals_ref, idx_ref):
    pre = lax.dot_general(
        x_ref[...], w_ref[...], (((1,), (1,)), ((), ())),
        preferred_element_type=jnp.float32,
    ) + b_ref[...]
    iota = lax.broadcasted_iota(jnp.int32, pre.shape, 1)
    v = pre
    for k in range(K):
        m = jnp.max(v, axis=1, keepdims=True)
        cand = jnp.where(v >= m, iota, SAE)
        am = jnp.min(cand, axis=1, keepdims=True)
        vals_ref[:, k:k + 1] = m
        idx_ref[:, k:k + 1] = am
        if k + 1 < K:
            v = jnp.where(iota == am, NEG, v)


def _run_topk(x, w_enc, b_enc):
    vals_p, idx_p = pl.pallas_call(
        _topk_body,
        grid=(B // BR,),
        in_specs=[
            pl.BlockSpec((BR, D), lambda i: (i, 0)),
            pl.BlockSpec((SAE, D), lambda i: (0, 0)),
            pl.BlockSpec((1, SAE), lambda i: (0, 0)),
        ],
        out_specs=[
            pl.BlockSpec((BR, 128), lambda i: (i, 0)),
            pl.BlockSpec((BR, 128), lambda i: (i, 0)),
        ],
        out_shape=[
            jax.ShapeDtypeStruct((B, 128), jnp.float32),
            jax.ShapeDtypeStruct((B, 128), jnp.int32),
        ],
    )(x, w_enc, b_enc.reshape(1, SAE))
    return vals_p[:, :K], idx_p[:, :K]


GPB = 8
AROWS = SAE // GPB
AL = GPB * L
SROWS = AROWS // NS


def _sc_body(x_hbm, idx_hbm, vals_hbm,
             we1_hbm, be1_hbm, we2_hbm, be2_hbm,
             wd0_hbm, wd1_hbm, wd2_hbm, bias_hbm,
             recon_hbm, acc_hbm,
             idx_v, idxp_v, idx8_v, vals_v, b1_v, b2_v,
             e1_v, e2_v, d0_v, d1_v, d2_v,
             x_v, out_v, bias_v,
             cr0_v, cr1_v, cr2_v,
             stat_v, zbuf_v, acc_sh,
             sem, semd):
    cid = lax.axis_index("c")
    sid = lax.axis_index("s")
    wid = sid * NC + cid
    row0 = wid * ROWS_PER_W
    lane = lax.broadcasted_iota(jnp.int32, (L,), 0)
    zvec = jnp.zeros((L,), jnp.float32)

    def zstep(i, _):
        for gg in range(GPB):
            zbuf_v[i, pl.ds(gg * L, L)] = zvec
        return 0
    lax.fori_loop(0, SROWS, zstep, 0)
    pltpu.sync_copy(zbuf_v, acc_sh.at[pl.ds(sid * SROWS, SROWS)])
    plsc.subcore_barrier()

    pltpu.sync_copy(bias_hbm, bias_v)

    def _sget(ref, e):
        g = (e // L) * L
        return ref[pl.ds(g, L)][e - g]

    def _rbf(v):
        u = lax.bitcast_convert_type(v, jnp.int32)
        r = (u + 0x8000) & jnp.int32(-65536)
        return lax.bitcast_convert_type(r, jnp.float32)

    def chunk_body(c, carry):
        rbase = row0 + c * RCH
        ebase = rbase * K
        pltpu.sync_copy(idx_hbm.at[pl.ds(ebase, ECH)], idx_v)
        pltpu.sync_copy(idx_hbm.at[pl.ds(ebase, ECH)], idxp_v.at[pl.ds(0, ECH)])
        pltpu.sync_copy(vals_hbm.at[pl.ds(ebase, ECH)], vals_v.at[pl.ds(0, ECH)])
        idx8_v[pl.ds(0, L)] = lax.shift_right_logical(idxp_v[pl.ds(0, L)], 3)
        idx8_v[pl.ds(ECH - L, L)] = lax.shift_right_logical(
            idxp_v[pl.ds(ECH - L, L)], 3)
        enc_cps = [
            pltpu.async_copy(we1_hbm.at[idx_v], e1_v, sem),
            pltpu.async_copy(we2_hbm.at[idx_v], e2_v, sem),
            pltpu.async_copy(be1_hbm.at[idx_v], b1_v.at[pl.ds(0, ECH)], sem),
            pltpu.async_copy(be2_hbm.at[idx_v], b2_v.at[pl.ds(0, ECH)], sem),
            pltpu.async_copy(x_hbm.at[pl.ds(rbase, RCH)], x_v, sem),
        ]
        dec_cps = [
            pltpu.async_copy(wd0_hbm.at[idx_v], d0_v, semd),
            pltpu.async_copy(wd1_hbm.at[idx_v], d1_v, semd),
            pltpu.async_copy(wd2_hbm.at[idx_v], d2_v, semd),
        ]
        for cp in enc_cps:
            cp.wait()

        for e in range(ECH):
            r = e // K

            def dstep(i, accs):
                a1, a2 = accs
                xs = _rbf(x_v[r, pl.ds(i * L, L)])
                return (a1 + _rbf(e1_v[e, pl.ds(i * L, L)]) * xs,
                        a2 + _rbf(e2_v[e, pl.ds(i * L, L)]) * xs)

            z = jnp.zeros((L,), jnp.float32)
            a1, a2 = lax.fori_loop(0, D // L, dstep, (z, z))
            p1 = jnp.sum(a1) + _sget(b1_v, e)
            p2 = jnp.sum(a2) + _sget(b2_v, e)
            v = _sget(vals_v, e)
            nz = v != 0.0
            m1 = jnp.where(nz, p1, 0.0)
            m2 = jnp.where(nz, p2, 0.0)
            w = m1 > m2
            f1 = jnp.where(w, m1, 0.0)
            f2 = jnp.where(w, 0.0, m2)
            cr0_v[e, :] = _rbf(jnp.full((L,), v))
            cr1_v[e, :] = _rbf(jnp.full((L,), f1))
            cr2_v[e, :] = _rbf(jnp.full((L,), f2))
            sv = jnp.full((L,), jnp.where(nz, v, 1.0))
            f1v = jnp.full((L,), f1)
            f2v = jnp.full((L,), f2)
            c1v = jnp.where(f1v != 0.0, 1.0, 0.0)
            c2v = jnp.where(f2v != 0.0, 1.0, 0.0)
            num = jnp.where(lane == 2, f1v, jnp.where(lane == 4, f2v, 0.0))
            cnts = jnp.where(lane == 0, 1.0,
                   jnp.where(lane == 1, c1v,
                   jnp.where(lane == 3, c2v, 0.0)))
            stat = cnts + num / sv
            g = _sget(idxp_v, e) & (GPB - 1)
            for gg in range(GPB):
                stat_v[e, pl.ds(gg * L, L)] = jnp.where(g == gg, stat, zvec)

        for cp in dec_cps:
            cp.wait()

        for r in range(RCH):
            coefs = []
            for k in range(K):
                e = r * K + k
                coefs.append((cr0_v[e, :], cr1_v[e, :], cr2_v[e, :]))

            def rstep(i, _):
                acc = bias_v[pl.ds(i * L, L)]
                for k in range(K):
                    e = r * K + k
                    cc0, cc1, cc2 = coefs[k]
                    acc = acc + cc0 * _rbf(d0_v[e, pl.ds(i * L, L)])
                    acc = acc + cc1 * _rbf(d1_v[e, pl.ds(i * L, L)])
                    acc = acc + cc2 * _rbf(d2_v[e, pl.ds(i * L, L)])
                out_v[r, pl.ds(i * L, L)] = acc
                return 0

            lax.fori_loop(0, D // L, rstep, 0)

        pltpu.sync_copy(out_v, recon_hbm.at[pl.ds(rbase, RCH)])
        pltpu.sync_copy(stat_v, acc_sh.at[idx8_v], add=True)
        return carry

    lax.fori_loop(0, NCHUNK, chunk_body, 0)

    plsc.subcore_barrier()
    pltpu.sync_copy(acc_sh.at[pl.ds(sid * SROWS, SROWS)],
                    acc_hbm.at[cid, pl.ds(sid * SROWS, SROWS)])


def _run_sc(x, idx_flat, vals_flat, w_enc1, b_enc1, w_enc2, b_enc2,
            wd0t, wd1t, wd2t, bias_sum):
    mesh = plsc.VectorSubcoreMesh(core_axis_name="c", subcore_axis_name="s",
                                  num_cores=NC, num_subcores=NS)
    f = pl.kernel(
        _sc_body,
        out_type=[
            jax.ShapeDtypeStruct((B, D), jnp.float32),
            jax.ShapeDtypeStruct((NC, AROWS, AL), jnp.float32),
        ],
        mesh=mesh,
        scratch_types=[
            pltpu.VMEM((ECH,), jnp.int32),
            pltpu.VMEM((ECH + L,), jnp.int32),
            pltpu.VMEM((ECH,), jnp.int32),
            pltpu.VMEM((ECH + L,), jnp.float32),
            pltpu.VMEM((ECH + L,), jnp.float32),
            pltpu.VMEM((ECH + L,), jnp.float32),
            pltpu.VMEM((ECH, D), jnp.float32),
            pltpu.VMEM((ECH, D), jnp.float32),
            pltpu.VMEM((ECH, D), jnp.float32),
            pltpu.VMEM((ECH, D), jnp.float32),
            pltpu.VMEM((ECH, D), jnp.float32),
            pltpu.VMEM((RCH, D), jnp.float32),
            pltpu.VMEM((RCH, D), jnp.float32),
            pltpu.VMEM((D,), jnp.float32),
            pltpu.VMEM((ECH, L), jnp.float32),
            pltpu.VMEM((ECH, L), jnp.float32),
            pltpu.VMEM((ECH, L), jnp.float32),
            pltpu.VMEM((ECH, AL), jnp.float32),
            pltpu.VMEM((SROWS, AL), jnp.float32),
            pltpu.VMEM_SHARED((AROWS, AL), jnp.float32),
            pltpu.SemaphoreType.DMA,
            pltpu.SemaphoreType.DMA,
        ],
        compiler_params=pltpu.CompilerParams(needs_layout_passes=False),
    )
    return f(x, idx_flat, vals_flat, w_enc1, b_enc1, w_enc2, b_enc2,
             wd0t, wd1t, wd2t, bias_sum)


def _finale_body(a0_ref, a1_ref, r1_ref, r2_ref,
                 r1o_ref, r2o_ref, live_ref):
    s = a0_ref[...] + a1_ref[...]
    cnt_p = s[:, 0:1]
    cnt1 = s[:, 1:2]
    sum1 = s[:, 2:3]
    cnt2 = s[:, 3:4]
    sum2 = s[:, 4:5]

    def ema(cnt, ssum, old):
        mean = ssum / jnp.maximum(cnt, 1.0)
        upd = (1.0 - EMA_COEFF * cnt) * old + EMA_COEFF * mean * cnt
        return jnp.where(cnt > 0.0, upd, old)

    r1o_ref[...] = ema(cnt1, sum1, r1_ref[...])
    r2o_ref[...] = ema(cnt2, sum2, r2_ref[...])

    lp = jnp.sum((cnt_p > 0.0).astype(jnp.float32))
    l1 = jnp.sum((cnt1 > 0.0).astype(jnp.float32))
    l2 = jnp.sum((cnt2 > 0.0).astype(jnp.float32))
    lanevec = lax.broadcasted_iota(jnp.int32, (1, 128), 1)
    live_ref[...] = jnp.where(lanevec == 0, lp,
                              jnp.where(lanevec == 1, l1,
                                        jnp.where(lanevec == 2, l2, 0.0)))


def _run_finale(acc, ratios1, ratios2):
    r1o, r2o, live = pl.pallas_call(
        _finale_body,
        out_shape=[
            jax.ShapeDtypeStruct((SAE, 1), jnp.float32),
            jax.ShapeDtypeStruct((SAE, 1), jnp.float32),
            jax.ShapeDtypeStruct((1, 128), jnp.float32),
        ],
    )(acc[0], acc[1], ratios1.reshape(SAE, 1), ratios2.reshape(SAE, 1))
    return r1o, r2o, live


def kernel(model_activations, W_enc, b_enc, W_dec, b_dec,
           W_enc1, b_enc1, W_dec1, b_dec1,
           W_enc2, b_enc2, W_dec2, b_dec2,
           child1_parent_ratios, child2_parent_ratios):
    x = model_activations
    topk_vals, topk_idx = _run_topk(x, W_enc, b_enc)
    idx_flat = topk_idx.reshape(-1)
    vals_flat = topk_vals.reshape(-1)

    wd0t = W_dec.T
    wd1t = W_dec1.T
    wd2t = W_dec2.T
    bias_sum = b_dec + b_dec1 + b_dec2

    recon, acc = _run_sc(
        x, idx_flat, vals_flat, W_enc1, b_enc1, W_enc2, b_enc2,
        wd0t, wd1t, wd2t, bias_sum)

    r1o, r2o, live = _run_finale(acc.reshape(NC, SAE, L),
                                 child1_parent_ratios,
                                 child2_parent_ratios)

    live_counts = live[0, :3].astype(jnp.int32)
    return (recon, live_counts,
            r1o.reshape(SAE), r2o.reshape(SAE))
